# baseline (speedup 1.0000x reference)
import jax
import jax.numpy as jnp
from jax import lax
from jax.experimental import pallas as pl
from jax.experimental.pallas import tpu as pltpu
from jax.experimental.pallas import tpu_sc as plsc

NUM_CLASSES = 5
LAT = 16
BATCH = 16384
NROWS = 1000000
GW = 128
RPG = GW // LAT
NGROUPS = NROWS * LAT // GW
NC, NS, L = 2, 16, 16
NW = NC * NS
BPW = BATCH // NW
CHUNK = 64
NCHUNK = BPW // CHUNK
NBLK = CHUNK // L


def _body(user_hbm, movie_hbm, utmf_hbm, mtmf_hbm, ut_hbm, mt_hbm,
          fcw_hbm, fcb_hbm, out_hbm,
          idx_u, idx_m, gidx_u, gidx_m,
          umf_v, mmf_v, u_v, m_v, w_v, b_v, out_v, sem):
    wid = lax.axis_index("s") * NC + lax.axis_index("c")
    row0 = wid * NCHUNK

    pltpu.sync_copy(user_hbm.at[pl.ds(row0, NCHUNK)], idx_u)
    pltpu.sync_copy(movie_hbm.at[pl.ds(row0, NCHUNK)], idx_m)
    pltpu.sync_copy(fcw_hbm, w_v)
    pltpu.sync_copy(fcb_hbm, b_v)

    for j in range(NCHUNK):
        for v in range(CHUNK // L):
            s = pl.ds(v * L, L)
            gidx_u[j, s] = lax.shift_right_logical(idx_u[j, s], 3)
            gidx_m[j, s] = lax.shift_right_logical(idx_m[j, s], 3)

    lane = lax.iota(jnp.int32, L)
    bias = b_v[...]

    for j in range(NCHUNK):
        copies = [
            pltpu.async_copy(utmf_hbm.at[gidx_u.at[j]], umf_v, sem),
            pltpu.async_copy(mtmf_hbm.at[gidx_m.at[j]], mmf_v, sem),
            pltpu.async_copy(ut_hbm.at[gidx_u.at[j]], u_v, sem),
            pltpu.async_copy(mt_hbm.at[gidx_m.at[j]], m_v, sem),
        ]
        for c in copies:
            c.wait()

        def blk_body(blk, carry, j=j):
            sb = pl.ds(blk * L, L)
            su_vec = (idx_u[j, sb] & 7) * LAT
            sm_vec = (idx_m[j, sb] & 7) * LAT
            for l in range(L):
                slot = blk * L + l
                su = su_vec[l]
                sm = sm_vec[l]
                mf = umf_v[slot, pl.ds(su, LAT)] * mmf_v[slot, pl.ds(sm, LAT)]
                u = u_v[slot, pl.ds(su, LAT)]
                m = m_v[slot, pl.ds(sm, LAT)]
                acc = bias
                for c in range(NUM_CLASSES):
                    t = (mf * w_v[c, 0:LAT] + u * w_v[c, LAT:2 * LAT]
                         + m * w_v[c, 2 * LAT:3 * LAT])
                    s = jnp.sum(t)
                    acc = jnp.where(lane == c, acc + s, acc)
                out_v[j * CHUNK + slot, :] = acc
            return carry

        lax.fori_loop(0, NBLK, blk_body, 0)

    pltpu.sync_copy(out_v, out_hbm.at[pl.ds(wid * BPW, BPW)])


def kernel(user, movie, user_table_mf, movie_table_mf, user_table,
           movie_table, fc_w, fc_b):
    user2 = user.reshape(NW * NCHUNK, CHUNK)
    movie2 = movie.reshape(NW * NCHUNK, CHUNK)
    utmf3 = user_table_mf.reshape(NGROUPS, GW)
    mtmf3 = movie_table_mf.reshape(NGROUPS, GW)
    ut3 = user_table.reshape(NGROUPS, GW)
    mt3 = movie_table.reshape(NGROUPS, GW)
    fcb_pad = jnp.pad(fc_b, (0, L - NUM_CLASSES))
    run = pl.kernel(
        _body,
        out_type=jax.ShapeDtypeStruct((BATCH, L), jnp.float32),
        mesh=plsc.VectorSubcoreMesh(core_axis_name="c", subcore_axis_name="s"),
        compiler_params=pltpu.CompilerParams(needs_layout_passes=False,
                                             use_tc_tiling_on_sc=True),
        scratch_types=[
            pltpu.VMEM((NCHUNK, CHUNK), jnp.int32),
            pltpu.VMEM((NCHUNK, CHUNK), jnp.int32),
            pltpu.VMEM((NCHUNK, CHUNK), jnp.int32),
            pltpu.VMEM((NCHUNK, CHUNK), jnp.int32),
            pltpu.VMEM((CHUNK, GW), jnp.float32),
            pltpu.VMEM((CHUNK, GW), jnp.float32),
            pltpu.VMEM((CHUNK, GW), jnp.float32),
            pltpu.VMEM((CHUNK, GW), jnp.float32),
            pltpu.VMEM((NUM_CLASSES, 3 * LAT), jnp.float32),
            pltpu.VMEM((L,), jnp.float32),
            pltpu.VMEM((BPW, L), jnp.float32),
            pltpu.SemaphoreType.DMA,
        ],
    )
    out_pad = run(user2, movie2, utmf3, mtmf3, ut3, mt3, fc_w, fcb_pad)
    return out_pad[:, :NUM_CLASSES]
